# baseline (speedup 1.0000x reference)
import functools

import jax
import jax.numpy as jnp
from jax import lax
from jax.experimental import pallas as pl
from jax.experimental.pallas import tpu as pltpu

ROWS = 128
COLS = 100000
RB = 8
CB = 4096
NC = (COLS + CB - 1) // CB
NR = ROWS // RB

_TINY = 1.1754943508222875e-38
_NEG_INF = float("-inf")


def _u32(v):
    return jnp.uint32(v)


def _rotl(x, d):
    return lax.shift_left(x, _u32(d)) | lax.shift_right_logical(x, _u32(32 - d))


def _threefry_gumbel(idx_u32):
    k0 = _u32(0)
    k1 = _u32(42)
    k2 = _u32(0x1BD11BDA) ^ k0 ^ k1
    ks = [k0, k1, k2]
    rot = [[13, 15, 26, 6], [17, 29, 16, 24]]
    x0 = jnp.zeros_like(idx_u32) + k0
    x1 = idx_u32 + k1
    for g in range(5):
        for r in rot[g % 2]:
            x0 = x0 + x1
            x1 = _rotl(x1, r)
            x1 = x1 ^ x0
        x0 = x0 + ks[(g + 1) % 3]
        x1 = x1 + ks[(g + 2) % 3] + _u32(g + 1)
    bits = x0 ^ x1
    fbits = lax.shift_right_logical(bits, _u32(9)) | _u32(0x3F800000)
    f = lax.bitcast_convert_type(fbits, jnp.float32) - jnp.float32(1.0)
    u = jnp.maximum(_TINY, f)
    return -jnp.log(-jnp.log(u))


def _race_kernel(x_ref, draw_ref, logp_ref, m_ref, s_ref, b_ref, xat_ref, idx_ref):
    r = pl.program_id(0)
    c = pl.program_id(1)

    @pl.when(c == 0)
    def _init():
        m_ref[...] = jnp.full_like(m_ref, _NEG_INF)
        s_ref[...] = jnp.zeros_like(s_ref)
        b_ref[...] = jnp.full_like(b_ref, _NEG_INF)
        xat_ref[...] = jnp.zeros_like(xat_ref)
        idx_ref[...] = jnp.zeros_like(idx_ref)

    x = x_ref[...]
    cols = c * CB + lax.broadcasted_iota(jnp.int32, (RB, CB), 1)
    valid = cols < COLS
    rows = r * RB + lax.broadcasted_iota(jnp.int32, (RB, CB), 0)
    flat = (rows * COLS + cols).astype(jnp.uint32)

    g = _threefry_gumbel(flat)
    xm = jnp.where(valid, x, _NEG_INF)
    score = jnp.where(valid, g + x, _NEG_INF)

    m_old = m_ref[:, 0:1]
    bx = jnp.max(xm, axis=1, keepdims=True)
    m_new = jnp.maximum(m_old, bx)
    s_ref[:, 0:1] = s_ref[:, 0:1] * jnp.exp(m_old - m_new) + jnp.sum(
        jnp.exp(xm - m_new), axis=1, keepdims=True
    )
    m_ref[:, 0:1] = m_new

    bs = jnp.max(score, axis=1, keepdims=True)
    eq = score == bs
    idx_blk = jnp.min(jnp.where(eq, cols, jnp.int32(2**31 - 1)), axis=1, keepdims=True)
    x_blk = jnp.max(jnp.where(cols == idx_blk, xm, _NEG_INF), axis=1, keepdims=True)
    upd = bs > b_ref[:, 0:1]
    b_ref[:, 0:1] = jnp.where(upd, bs, b_ref[:, 0:1])
    idx_ref[:, 0:1] = jnp.where(upd, idx_blk, idx_ref[:, 0:1])
    xat_ref[:, 0:1] = jnp.where(upd, x_blk, xat_ref[:, 0:1])

    @pl.when(c == NC - 1)
    def _finish():
        logz = m_ref[:, 0:1] + jnp.log(s_ref[:, 0:1])
        logp_ref[...] = xat_ref[:, 0:1] - logz
        draw_ref[...] = idx_ref[:, 0:1]


def _onehot_kernel(draw_ref, out_ref):
    c = pl.program_id(1)
    cols = c * CB + lax.broadcasted_iota(jnp.int32, (RB, CB), 1)
    out_ref[...] = jnp.where(cols == draw_ref[...], jnp.float32(1.0), jnp.float32(0.0))


@jax.jit
def kernel(logits):
    draw, logp = pl.pallas_call(
        _race_kernel,
        grid=(NR, NC),
        in_specs=[pl.BlockSpec((RB, CB), lambda r, c: (r, c))],
        out_specs=[
            pl.BlockSpec((RB, 1), lambda r, c: (r, 0)),
            pl.BlockSpec((RB, 1), lambda r, c: (r, 0)),
        ],
        out_shape=[
            jax.ShapeDtypeStruct((ROWS, 1), jnp.int32),
            jax.ShapeDtypeStruct((ROWS, 1), jnp.float32),
        ],
        scratch_shapes=[
            pltpu.VMEM((RB, 128), jnp.float32),
            pltpu.VMEM((RB, 128), jnp.float32),
            pltpu.VMEM((RB, 128), jnp.float32),
            pltpu.VMEM((RB, 128), jnp.float32),
            pltpu.VMEM((RB, 128), jnp.int32),
        ],
    )(logits)

    samples = pl.pallas_call(
        _onehot_kernel,
        grid=(NR, NC),
        in_specs=[pl.BlockSpec((RB, 1), lambda r, c: (r, 0))],
        out_specs=pl.BlockSpec((RB, CB), lambda r, c: (r, c)),
        out_shape=jax.ShapeDtypeStruct((ROWS, COLS), jnp.float32),
    )(draw)

    return samples, logp.reshape(ROWS)


# speedup vs baseline: 1.4381x; 1.4381x over previous
import jax
import jax.numpy as jnp
from jax import lax
from jax.experimental import pallas as pl
from jax.experimental.pallas import tpu as pltpu

ROWS = 128
COLS = 100000
RB = 8
CB = 8192
CH = 512
NC = (COLS + CB - 1) // CB
NR = ROWS // RB

_TINY = 1.1754943508222875e-38
_NEG_INF = float("-inf")
_INT_MAX = 2**31 - 1


def _u32(v):
    return jnp.uint32(v)


def _rotl(x, d):
    return lax.shift_left(x, _u32(d)) | lax.shift_right_logical(x, _u32(32 - d))


def _threefry_gumbel(idx_u32):
    k0 = _u32(0)
    k1 = _u32(42)
    k2 = _u32(0x1BD11BDA) ^ k0 ^ k1
    ks = [k0, k1, k2]
    rot = [[13, 15, 26, 6], [17, 29, 16, 24]]
    x0 = jnp.zeros_like(idx_u32) + k0
    x1 = idx_u32 + k1
    for g in range(5):
        for r in rot[g % 2]:
            x0 = x0 + x1
            x1 = _rotl(x1, r)
            x1 = x1 ^ x0
        x0 = x0 + ks[(g + 1) % 3]
        x1 = x1 + ks[(g + 2) % 3] + _u32(g + 1)
    bits = x0 ^ x1
    fbits = lax.shift_right_logical(bits, _u32(9)) | _u32(0x3F800000)
    f = lax.bitcast_convert_type(fbits, jnp.float32) - jnp.float32(1.0)
    u = jnp.maximum(jnp.float32(_TINY), f)
    return -jnp.log(-jnp.log(u))


def _race_kernel(x_ref, draw_ref, logp_ref, mv_ref, sv_ref, bv_ref, xv_ref, iv_ref):
    r = pl.program_id(0)
    c = pl.program_id(1)

    @pl.when(c == 0)
    def _init():
        mv_ref[...] = jnp.full_like(mv_ref, _NEG_INF)
        sv_ref[...] = jnp.zeros_like(sv_ref)
        bv_ref[...] = jnp.full_like(bv_ref, _NEG_INF)
        xv_ref[...] = jnp.zeros_like(xv_ref)
        iv_ref[...] = jnp.zeros_like(iv_ref)

    lane = lax.broadcasted_iota(jnp.int32, (RB, CH), 1)
    rowoff = (r * RB + lax.broadcasted_iota(jnp.int32, (RB, CH), 0)) * COLS + lane

    m_v = mv_ref[...]
    s_v = sv_ref[...]
    b_v = bv_ref[...]
    x_v = xv_ref[...]
    i_v = iv_ref[...]

    for k in range(CB // CH):
        base = c * CB + k * CH
        xc = x_ref[:, k * CH : (k + 1) * CH]
        cols = base + lane
        valid = cols < COLS
        g = _threefry_gumbel((rowoff + base).astype(jnp.uint32))
        xm = jnp.where(valid, xc, _NEG_INF)
        score = jnp.where(valid, g + xc, _NEG_INF)
        upd = score > b_v
        b_v = jnp.where(upd, score, b_v)
        i_v = jnp.where(upd, cols, i_v)
        x_v = jnp.where(upd, xm, x_v)
        m_new = jnp.maximum(m_v, xm)
        s_v = s_v * jnp.exp(m_v - m_new) + jnp.exp(xm - m_new)
        m_v = m_new

    mv_ref[...] = m_v
    sv_ref[...] = s_v
    bv_ref[...] = b_v
    xv_ref[...] = x_v
    iv_ref[...] = i_v

    @pl.when(c == NC - 1)
    def _finish():
        m = jnp.max(m_v, axis=1, keepdims=True)
        s = jnp.sum(s_v * jnp.exp(m_v - m), axis=1, keepdims=True)
        logz = m + jnp.log(s)
        best = jnp.max(b_v, axis=1, keepdims=True)
        eq = b_v == best
        idx = jnp.min(jnp.where(eq, i_v, _INT_MAX), axis=1, keepdims=True)
        xat = jnp.max(
            jnp.where(eq & (i_v == idx), x_v, _NEG_INF), axis=1, keepdims=True
        )
        logp_ref[...] = xat - logz
        draw_ref[...] = idx


def _onehot_kernel(draw_ref, out_ref):
    c = pl.program_id(1)
    cols = c * CB + lax.broadcasted_iota(jnp.int32, (RB, CB), 1)
    out_ref[...] = jnp.where(cols == draw_ref[...], jnp.float32(1.0), jnp.float32(0.0))


@jax.jit
def kernel(logits):
    draw, logp = pl.pallas_call(
        _race_kernel,
        grid=(NR, NC),
        in_specs=[pl.BlockSpec((RB, CB), lambda r, c: (r, c))],
        out_specs=[
            pl.BlockSpec((RB, 1), lambda r, c: (r, 0)),
            pl.BlockSpec((RB, 1), lambda r, c: (r, 0)),
        ],
        out_shape=[
            jax.ShapeDtypeStruct((ROWS, 1), jnp.int32),
            jax.ShapeDtypeStruct((ROWS, 1), jnp.float32),
        ],
        scratch_shapes=[
            pltpu.VMEM((RB, CH), jnp.float32),
            pltpu.VMEM((RB, CH), jnp.float32),
            pltpu.VMEM((RB, CH), jnp.float32),
            pltpu.VMEM((RB, CH), jnp.float32),
            pltpu.VMEM((RB, CH), jnp.int32),
        ],
    )(logits)

    samples = pl.pallas_call(
        _onehot_kernel,
        grid=(NR, NC),
        in_specs=[pl.BlockSpec((RB, 1), lambda r, c: (r, 0))],
        out_specs=pl.BlockSpec((RB, CB), lambda r, c: (r, c)),
        out_shape=jax.ShapeDtypeStruct((ROWS, COLS), jnp.float32),
    )(draw)

    return samples, logp.reshape(ROWS)


# speedup vs baseline: 1.4896x; 1.0357x over previous
import jax
import jax.numpy as jnp
from jax import lax
from jax.experimental import pallas as pl
from jax.experimental.pallas import tpu as pltpu

ROWS = 128
COLS = 100000
RB = 8
CB = 8192
CH = 256
NCHUNK = CB // CH
NC = (COLS + CB - 1) // CB
NR = ROWS // RB

_TINY = 1.1754943508222875e-38
_NEG_INF = float("-inf")
_INT_MAX = 2**31 - 1


def _u32(v):
    return jnp.uint32(v)


def _rotl(x, d):
    return lax.shift_left(x, _u32(d)) | lax.shift_right_logical(x, _u32(32 - d))


def _threefry_gumbel(x1):
    k1 = _u32(42)
    k2 = _u32(0x1BD11BDA) ^ k1
    ks = [_u32(0), k1, k2]
    rot = [[13, 15, 26, 6], [17, 29, 16, 24]]
    x0 = x1
    for g in range(5):
        for i, r in enumerate(rot[g % 2]):
            if not (g == 0 and i == 0):
                x0 = x0 + x1
            x1 = _rotl(x1, r)
            x1 = x1 ^ x0
        x0 = x0 + ks[(g + 1) % 3]
        x1 = x1 + ks[(g + 2) % 3] + _u32(g + 1)
    bits = x0 ^ x1
    fbits = lax.shift_right_logical(bits, _u32(9)) | _u32(0x3F800000)
    f = lax.bitcast_convert_type(fbits, jnp.float32) - jnp.float32(1.0)
    u = jnp.maximum(jnp.float32(_TINY), f)
    return -jnp.log(-jnp.log(u))


def _race_kernel(x_ref, draw_ref, logp_ref, mv_ref, sv_ref, bv_ref, kv_ref):
    r = pl.program_id(0)
    c = pl.program_id(1)

    @pl.when(c == 0)
    def _init():
        mv_ref[...] = jnp.full_like(mv_ref, _NEG_INF)
        sv_ref[...] = jnp.zeros_like(sv_ref)
        bv_ref[...] = jnp.full_like(bv_ref, _NEG_INF)
        kv_ref[...] = jnp.zeros_like(kv_ref)

    lane = lax.broadcasted_iota(jnp.int32, (RB, CH), 1)
    rowoff42 = (r * RB + lax.broadcasted_iota(jnp.int32, (RB, CH), 0)) * COLS + lane + 42

    def run(masked):
        m_v = mv_ref[...]
        s_v = sv_ref[...]
        b_v = bv_ref[...]
        k_v = kv_ref[...]
        for k in range(NCHUNK):
            base = c * CB + k * CH
            xc = x_ref[:, k * CH : (k + 1) * CH]
            g = _threefry_gumbel((rowoff42 + base).astype(jnp.uint32))
            if masked:
                valid = (base + lane) < COLS
                xm = jnp.where(valid, xc, _NEG_INF)
                score = jnp.where(valid, g + xc, _NEG_INF)
            else:
                xm = xc
                score = g + xc
            upd = score > b_v
            b_v = jnp.where(upd, score, b_v)
            k_v = jnp.where(upd, c * NCHUNK + k, k_v)
            m_new = jnp.maximum(m_v, xm)
            s_v = s_v * jnp.exp(m_v - m_new) + jnp.exp(xm - m_new)
            m_v = m_new
        mv_ref[...] = m_v
        sv_ref[...] = s_v
        bv_ref[...] = b_v
        kv_ref[...] = k_v

    pl.when(c < NC - 1)(lambda: run(False))
    pl.when(c == NC - 1)(lambda: run(True))

    @pl.when(c == NC - 1)
    def _finish():
        m_v = mv_ref[...]
        s_v = sv_ref[...]
        b_v = bv_ref[...]
        k_v = kv_ref[...]
        m = jnp.max(m_v, axis=1, keepdims=True)
        s = jnp.sum(s_v * jnp.exp(m_v - m), axis=1, keepdims=True)
        logz = m + jnp.log(s)
        best = jnp.max(b_v, axis=1, keepdims=True)
        eq = b_v == best
        cols_v = k_v * CH + lane
        idx = jnp.min(jnp.where(eq, cols_v, _INT_MAX), axis=1, keepdims=True)
        row_ids = r * RB + lax.broadcasted_iota(jnp.int32, (RB, 1), 0)
        g_at = _threefry_gumbel((row_ids * COLS + idx + 42).astype(jnp.uint32))
        logp_ref[...] = (best - g_at) - logz
        draw_ref[...] = idx


def _onehot_kernel(draw_ref, out_ref):
    c = pl.program_id(1)
    cols = c * CB + lax.broadcasted_iota(jnp.int32, (RB, CB), 1)
    out_ref[...] = jnp.where(cols == draw_ref[...], jnp.float32(1.0), jnp.float32(0.0))


@jax.jit
def kernel(logits):
    draw, logp = pl.pallas_call(
        _race_kernel,
        grid=(NR, NC),
        in_specs=[pl.BlockSpec((RB, CB), lambda r, c: (r, c))],
        out_specs=[
            pl.BlockSpec((RB, 1), lambda r, c: (r, 0)),
            pl.BlockSpec((RB, 1), lambda r, c: (r, 0)),
        ],
        out_shape=[
            jax.ShapeDtypeStruct((ROWS, 1), jnp.int32),
            jax.ShapeDtypeStruct((ROWS, 1), jnp.float32),
        ],
        scratch_shapes=[
            pltpu.VMEM((RB, CH), jnp.float32),
            pltpu.VMEM((RB, CH), jnp.float32),
            pltpu.VMEM((RB, CH), jnp.float32),
            pltpu.VMEM((RB, CH), jnp.int32),
        ],
    )(logits)

    samples = pl.pallas_call(
        _onehot_kernel,
        grid=(NR, NC),
        in_specs=[pl.BlockSpec((RB, 1), lambda r, c: (r, 0))],
        out_specs=pl.BlockSpec((RB, CB), lambda r, c: (r, c)),
        out_shape=jax.ShapeDtypeStruct((ROWS, COLS), jnp.float32),
    )(draw)

    return samples, logp.reshape(ROWS)


# speedup vs baseline: 1.7337x; 1.1639x over previous
import jax
import jax.numpy as jnp
from jax import lax
from jax.experimental import pallas as pl
from jax.experimental.pallas import tpu as pltpu

ROWS = 128
COLS = 100000
RB = 8
CH = 256
NCHUNK = 23
NC = 17
NR = ROWS // RB
STEP = NCHUNK * CH

_TINY = 1.1754943508222875e-38
_NEG_INF = float("-inf")
_INT_MAX = 2**31 - 1


def _u32(v):
    return jnp.uint32(v)


def _rotl(x, d):
    return lax.shift_left(x, _u32(d)) | lax.shift_right_logical(x, _u32(32 - d))


def _threefry_gumbel(x1):
    k1 = _u32(42)
    k2 = _u32(0x1BD11BDA) ^ k1
    ks = [_u32(0), k1, k2]
    rot = [[13, 15, 26, 6], [17, 29, 16, 24]]
    x0 = x1
    for g in range(5):
        for i, r in enumerate(rot[g % 2]):
            if not (g == 0 and i == 0):
                x0 = x0 + x1
            x1 = _rotl(x1, r)
            x1 = x1 ^ x0
        x0 = x0 + ks[(g + 1) % 3]
        x1 = x1 + ks[(g + 2) % 3] + _u32(g + 1)
    bits = x0 ^ x1
    fbits = lax.shift_right_logical(bits, _u32(9)) | _u32(0x3F800000)
    f = lax.bitcast_convert_type(fbits, jnp.float32) - jnp.float32(1.0)
    u = jnp.maximum(jnp.float32(_TINY), f)
    return -jnp.log(-jnp.log(u))


def _race_kernel(x_ref, draw_ref, logp_ref, mv_ref, sv_ref, bv_ref, kv_ref):
    r = pl.program_id(0)
    c = pl.program_id(1)

    @pl.when(c == 0)
    def _init():
        mv_ref[...] = jnp.full_like(mv_ref, _NEG_INF)
        sv_ref[...] = jnp.zeros_like(sv_ref)
        bv_ref[...] = jnp.full_like(bv_ref, _NEG_INF)
        kv_ref[...] = jnp.zeros_like(kv_ref)

    lane = lax.broadcasted_iota(jnp.int32, (RB, CH), 1)
    rowoff42 = (r * RB + lax.broadcasted_iota(jnp.int32, (RB, CH), 0)) * COLS + lane + 42

    def run(masked):
        m_v = mv_ref[...]
        s_v = sv_ref[...]
        b_v = bv_ref[...]
        k_v = kv_ref[...]
        for k in range(NCHUNK):
            tail = masked and k == NCHUNK - 1
            if masked:
                base = min((NC - 1) * STEP + k * CH, COLS - CH)
            else:
                base = c * STEP + k * CH
            xc = x_ref[:, pl.ds(base, CH)]
            g = _threefry_gumbel((rowoff42 + base).astype(jnp.uint32))
            if tail:
                valid = lane >= ((NC - 1) * STEP + k * CH) - (COLS - CH)
                xm = jnp.where(valid, xc, _NEG_INF)
                score = jnp.where(valid, g + xc, _NEG_INF)
            else:
                xm = xc
                score = g + xc
            upd = score > b_v
            b_v = jnp.where(upd, score, b_v)
            k_v = jnp.where(upd, c * NCHUNK + k, k_v)
            m_new = jnp.maximum(m_v, xm)
            s_v = s_v * jnp.exp(m_v - m_new) + jnp.exp(xm - m_new)
            m_v = m_new
        mv_ref[...] = m_v
        sv_ref[...] = s_v
        bv_ref[...] = b_v
        kv_ref[...] = k_v

    pl.when(c < NC - 1)(lambda: run(False))
    pl.when(c == NC - 1)(lambda: run(True))

    @pl.when(c == NC - 1)
    def _finish():
        m_v = mv_ref[...]
        s_v = sv_ref[...]
        b_v = bv_ref[...]
        k_v = kv_ref[...]
        m = jnp.max(m_v, axis=1, keepdims=True)
        s = jnp.sum(s_v * jnp.exp(m_v - m), axis=1, keepdims=True)
        logz = m + jnp.log(s)
        best = jnp.max(b_v, axis=1, keepdims=True)
        eq = b_v == best
        cols_v = k_v * CH + lane
        tail_shift = (NC - 1) * STEP + (NCHUNK - 1) * CH - (COLS - CH)
        cols_v = jnp.where(k_v == NC * NCHUNK - 1, cols_v - tail_shift, cols_v)
        idx = jnp.min(jnp.where(eq, cols_v, _INT_MAX), axis=1, keepdims=True)
        row_ids = r * RB + lax.broadcasted_iota(jnp.int32, (RB, 1), 0)
        g_at = _threefry_gumbel((row_ids * COLS + idx + 42).astype(jnp.uint32))
        logp_ref[...] = (best - g_at) - logz
        draw_ref[...] = idx


def _onehot_kernel(draw_ref, out_ref):
    cols = lax.broadcasted_iota(jnp.int32, (RB, COLS), 1)
    out_ref[...] = jnp.where(cols == draw_ref[...], jnp.float32(1.0), jnp.float32(0.0))


@jax.jit
def kernel(logits):
    draw, logp = pl.pallas_call(
        _race_kernel,
        grid=(NR, NC),
        in_specs=[pl.BlockSpec((RB, COLS), lambda r, c: (r, 0))],
        out_specs=[
            pl.BlockSpec((RB, 1), lambda r, c: (r, 0)),
            pl.BlockSpec((RB, 1), lambda r, c: (r, 0)),
        ],
        out_shape=[
            jax.ShapeDtypeStruct((ROWS, 1), jnp.int32),
            jax.ShapeDtypeStruct((ROWS, 1), jnp.float32),
        ],
        scratch_shapes=[
            pltpu.VMEM((RB, CH), jnp.float32),
            pltpu.VMEM((RB, CH), jnp.float32),
            pltpu.VMEM((RB, CH), jnp.float32),
            pltpu.VMEM((RB, CH), jnp.int32),
        ],
    )(logits)

    samples = pl.pallas_call(
        _onehot_kernel,
        grid=(NR,),
        in_specs=[pl.BlockSpec((RB, 1), lambda r: (r, 0))],
        out_specs=pl.BlockSpec((RB, COLS), lambda r: (r, 0)),
        out_shape=jax.ShapeDtypeStruct((ROWS, COLS), jnp.float32),
    )(draw)

    return samples, logp.reshape(ROWS)


# speedup vs baseline: 1.9003x; 1.0961x over previous
import jax
import jax.numpy as jnp
from jax import lax
from jax.experimental import pallas as pl

ROWS = 128
COLS = 100000
RB = 8
CH = 256
NCHUNK = 391
NR = ROWS // RB

_TINY = 1.1754943508222875e-38
_NEG_INF = float("-inf")
_INT_MAX = 2**31 - 1


def _u32(v):
    return jnp.uint32(v)


def _rotl(x, d):
    return lax.shift_left(x, _u32(d)) | lax.shift_right_logical(x, _u32(32 - d))


def _threefry_gumbel(x1):
    k1 = _u32(42)
    k2 = _u32(0x1BD11BDA) ^ k1
    ks = [_u32(0), k1, k2]
    rot = [[13, 15, 26, 6], [17, 29, 16, 24]]
    x0 = x1
    for g in range(5):
        for i, r in enumerate(rot[g % 2]):
            if not (g == 0 and i == 0):
                x0 = x0 + x1
            x1 = _rotl(x1, r)
            x1 = x1 ^ x0
        x0 = x0 + ks[(g + 1) % 3]
        x1 = x1 + ks[(g + 2) % 3] + _u32(g + 1)
    bits = x0 ^ x1
    fbits = lax.shift_right_logical(bits, _u32(9)) | _u32(0x3F800000)
    f = lax.bitcast_convert_type(fbits, jnp.float32) - jnp.float32(1.0)
    u = jnp.maximum(jnp.float32(_TINY), f)
    return -jnp.log(-jnp.log(u))


def _race_kernel(x_ref, draw_ref, logp_ref):
    r = pl.program_id(0)

    lane = lax.broadcasted_iota(jnp.int32, (RB, CH), 1)
    rowoff42 = (r * RB + lax.broadcasted_iota(jnp.int32, (RB, CH), 0)) * COLS + lane + 42

    m_v = jnp.full((RB, CH), _NEG_INF, jnp.float32)
    s_v = jnp.zeros((RB, CH), jnp.float32)
    b_v = jnp.full((RB, CH), _NEG_INF, jnp.float32)
    k_v = jnp.zeros((RB, CH), jnp.int32)

    for k in range(NCHUNK):
        tail = k == NCHUNK - 1
        base = min(k * CH, COLS - CH)
        xc = x_ref[:, pl.ds(base, CH)]
        g = _threefry_gumbel((rowoff42 + base).astype(jnp.uint32))
        if tail:
            valid = lane >= (NCHUNK - 1) * CH - (COLS - CH)
            xm = jnp.where(valid, xc, _NEG_INF)
            score = jnp.where(valid, g + xc, _NEG_INF)
        else:
            xm = xc
            score = g + xc
        upd = score > b_v
        b_v = jnp.where(upd, score, b_v)
        k_v = jnp.where(upd, k, k_v)
        m_new = jnp.maximum(m_v, xm)
        s_v = s_v * jnp.exp(m_v - m_new) + jnp.exp(xm - m_new)
        m_v = m_new

    m = jnp.max(m_v, axis=1, keepdims=True)
    s = jnp.sum(s_v * jnp.exp(m_v - m), axis=1, keepdims=True)
    logz = m + jnp.log(s)
    best = jnp.max(b_v, axis=1, keepdims=True)
    eq = b_v == best
    cols_v = k_v * CH + lane
    tail_shift = (NCHUNK - 1) * CH - (COLS - CH)
    cols_v = jnp.where(k_v == NCHUNK - 1, cols_v - tail_shift, cols_v)
    idx = jnp.min(jnp.where(eq, cols_v, _INT_MAX), axis=1, keepdims=True)
    row_ids = r * RB + lax.broadcasted_iota(jnp.int32, (RB, 1), 0)
    g_at = _threefry_gumbel((row_ids * COLS + idx + 42).astype(jnp.uint32))
    logp_ref[...] = (best - g_at) - logz
    draw_ref[...] = idx


def _onehot_kernel(draw_ref, out_ref):
    cols = lax.broadcasted_iota(jnp.int32, (RB, COLS), 1)
    out_ref[...] = jnp.where(cols == draw_ref[...], jnp.float32(1.0), jnp.float32(0.0))


@jax.jit
def kernel(logits):
    draw, logp = pl.pallas_call(
        _race_kernel,
        grid=(NR,),
        in_specs=[pl.BlockSpec((RB, COLS), lambda r: (r, 0))],
        out_specs=[
            pl.BlockSpec((RB, 1), lambda r: (r, 0)),
            pl.BlockSpec((RB, 1), lambda r: (r, 0)),
        ],
        out_shape=[
            jax.ShapeDtypeStruct((ROWS, 1), jnp.int32),
            jax.ShapeDtypeStruct((ROWS, 1), jnp.float32),
        ],
    )(logits)

    samples = pl.pallas_call(
        _onehot_kernel,
        grid=(NR,),
        in_specs=[pl.BlockSpec((RB, 1), lambda r: (r, 0))],
        out_specs=pl.BlockSpec((RB, COLS), lambda r: (r, 0)),
        out_shape=jax.ShapeDtypeStruct((ROWS, COLS), jnp.float32),
    )(draw)

    return samples, logp.reshape(ROWS)


# speedup vs baseline: 1.9628x; 1.0329x over previous
import jax
import jax.numpy as jnp
from jax import lax
from jax.experimental import pallas as pl
from jax.experimental.pallas import tpu as pltpu

ROWS = 128
COLS = 100000
RB = 8
CH = 256
NCHUNK = 391
NR = ROWS // RB

_TINY = 1.1754943508222875e-38
_NEG_INF = float("-inf")
_INT_MAX = 2**31 - 1


def _u32(v):
    return jnp.uint32(v)


def _rotl(x, d):
    return lax.shift_left(x, _u32(d)) | lax.shift_right_logical(x, _u32(32 - d))


def _threefry_gumbel(x1):
    k1 = _u32(42)
    k2 = _u32(0x1BD11BDA) ^ k1
    ks = [_u32(0), k1, k2]
    rot = [[13, 15, 26, 6], [17, 29, 16, 24]]
    x0 = x1
    for g in range(5):
        for i, r in enumerate(rot[g % 2]):
            if not (g == 0 and i == 0):
                x0 = x0 + x1
            x1 = _rotl(x1, r)
            x1 = x1 ^ x0
        x0 = x0 + ks[(g + 1) % 3]
        x1 = x1 + ks[(g + 2) % 3] + _u32(g + 1)
    bits = x0 ^ x1
    fbits = lax.shift_right_logical(bits, _u32(9)) | _u32(0x3F800000)
    f = lax.bitcast_convert_type(fbits, jnp.float32) - jnp.float32(1.0)
    u = jnp.maximum(jnp.float32(_TINY), f)
    return -jnp.log(-jnp.log(u))


def _race_kernel(x_ref, draw_ref, logp_ref, samples_ref, prev_ref):
    r = pl.program_id(0)

    @pl.when(r > 0)
    def _write_prev():
        cols = lax.broadcasted_iota(jnp.int32, (RB, COLS), 1)
        samples_ref[...] = jnp.where(
            cols == prev_ref[...], jnp.float32(1.0), jnp.float32(0.0)
        )

    @pl.when(r < NR)
    def _race():
        _race_body(x_ref, draw_ref, logp_ref, prev_ref, r)


def _race_body(x_ref, draw_ref, logp_ref, prev_ref, r):
    lane = lax.broadcasted_iota(jnp.int32, (RB, CH), 1)
    rowoff42 = (r * RB + lax.broadcasted_iota(jnp.int32, (RB, CH), 0)) * COLS + lane + 42

    m_v = jnp.full((RB, CH), _NEG_INF, jnp.float32)
    s_v = jnp.zeros((RB, CH), jnp.float32)
    b_v = jnp.full((RB, CH), _NEG_INF, jnp.float32)
    k_v = jnp.zeros((RB, CH), jnp.int32)

    for k in range(NCHUNK):
        tail = k == NCHUNK - 1
        base = min(k * CH, COLS - CH)
        xc = x_ref[:, pl.ds(base, CH)]
        g = _threefry_gumbel((rowoff42 + base).astype(jnp.uint32))
        if tail:
            valid = lane >= (NCHUNK - 1) * CH - (COLS - CH)
            xm = jnp.where(valid, xc, _NEG_INF)
            score = jnp.where(valid, g + xc, _NEG_INF)
        else:
            xm = xc
            score = g + xc
        upd = score > b_v
        b_v = jnp.where(upd, score, b_v)
        k_v = jnp.where(upd, k, k_v)
        m_new = jnp.maximum(m_v, xm)
        s_v = s_v * jnp.exp(m_v - m_new) + jnp.exp(xm - m_new)
        m_v = m_new

    m = jnp.max(m_v, axis=1, keepdims=True)
    s = jnp.sum(s_v * jnp.exp(m_v - m), axis=1, keepdims=True)
    logz = m + jnp.log(s)
    best = jnp.max(b_v, axis=1, keepdims=True)
    eq = b_v == best
    cols_v = k_v * CH + lane
    tail_shift = (NCHUNK - 1) * CH - (COLS - CH)
    cols_v = jnp.where(k_v == NCHUNK - 1, cols_v - tail_shift, cols_v)
    idx = jnp.min(jnp.where(eq, cols_v, _INT_MAX), axis=1, keepdims=True)
    row_ids = r * RB + lax.broadcasted_iota(jnp.int32, (RB, 1), 0)
    g_at = _threefry_gumbel((row_ids * COLS + idx + 42).astype(jnp.uint32))
    logp_ref[...] = (best - g_at) - logz
    draw_ref[...] = idx
    prev_ref[...] = idx


@jax.jit
def kernel(logits):
    draw, logp, samples = pl.pallas_call(
        _race_kernel,
        grid=(NR + 1,),
        in_specs=[
            pl.BlockSpec((RB, COLS), lambda r: (jnp.minimum(r, NR - 1), 0))
        ],
        out_specs=[
            pl.BlockSpec((RB, 1), lambda r: (jnp.minimum(r, NR - 1), 0)),
            pl.BlockSpec((RB, 1), lambda r: (jnp.minimum(r, NR - 1), 0)),
            pl.BlockSpec((RB, COLS), lambda r: (jnp.maximum(r - 1, 0), 0)),
        ],
        out_shape=[
            jax.ShapeDtypeStruct((ROWS, 1), jnp.int32),
            jax.ShapeDtypeStruct((ROWS, 1), jnp.float32),
            jax.ShapeDtypeStruct((ROWS, COLS), jnp.float32),
        ],
        scratch_shapes=[pltpu.VMEM((RB, 1), jnp.int32)],
    )(logits)

    return samples, logp.reshape(ROWS)
